# baseline (speedup 1.0000x reference)
import functools

import jax
import jax.numpy as jnp
from jax import lax
from jax.experimental import pallas as pl
from jax.experimental.pallas import tpu as pltpu
from jax.experimental.pallas import tpu_sc as plsc

_LANES = 16
_SUBCORES = 16
_CORES = 2
_BATCH = 128
_CHUNK = 8
_FEAT = 128
_ROWS_TC = 1000


def _mm_bias_body(x_ref, w_ref, b_ref, o_ref):
    o_ref[0] = (
        jnp.dot(x_ref[...], w_ref[0], preferred_element_type=jnp.float32)
        + b_ref[0, 0][None, :]
    )


def _mm_bias(x, w, b):
    n, k = x.shape
    m = w.shape[1]
    p = m // _FEAT
    return pl.pallas_call(
        _mm_bias_body,
        grid=(p, n // _ROWS_TC),
        in_specs=[
            pl.BlockSpec((_ROWS_TC, k), lambda j, i: (i, 0)),
            pl.BlockSpec((1, k, _FEAT), lambda j, i: (j, 0, 0)),
            pl.BlockSpec((1, 1, _FEAT), lambda j, i: (j, 0, 0)),
        ],
        out_specs=pl.BlockSpec((1, _ROWS_TC, _FEAT), lambda j, i: (j, i, 0)),
        out_shape=jax.ShapeDtypeStruct((p, n, _FEAT), jnp.float32),
    )(x, w.reshape(k, p, _FEAT).transpose(1, 0, 2), b.reshape(p, 1, _FEAT))


def _mm_enc_body(e_ref, x_ref, w_ref, b_ref, o_ref):
    acc = jnp.dot(x_ref[...], w_ref[0, 256:384],
                  preferred_element_type=jnp.float32)
    for u in range(2):
        acc += jnp.dot(e_ref[0, u] + e_ref[1, u],
                       w_ref[0, _FEAT * u:_FEAT * (u + 1)],
                       preferred_element_type=jnp.float32)
    o_ref[0] = acc + b_ref[0, 0][None, :]


def _mm_enc(enc_part, xq, wma, b):
    n = xq.shape[0]
    k = wma.shape[0]
    p = wma.shape[1] // _FEAT
    return pl.pallas_call(
        _mm_enc_body,
        grid=(p, n // _ROWS_TC),
        in_specs=[
            pl.BlockSpec((2, 2, _ROWS_TC, _FEAT), lambda j, i: (0, 0, i, 0)),
            pl.BlockSpec((_ROWS_TC, 128), lambda j, i: (i, 0)),
            pl.BlockSpec((1, k, _FEAT), lambda j, i: (j, 0, 0)),
            pl.BlockSpec((1, 1, _FEAT), lambda j, i: (j, 0, 0)),
        ],
        out_specs=pl.BlockSpec((1, _ROWS_TC, _FEAT), lambda j, i: (j, i, 0)),
        out_shape=jax.ShapeDtypeStruct((p, n, _FEAT), jnp.float32),
    )(enc_part, xq, wma.reshape(k, p, _FEAT).transpose(1, 0, 2),
      b.reshape(p, 1, _FEAT))


def _dec_body(m_ref, w_ref, b_ref, o_ref):
    acc = b_ref[0][None, :] + jnp.dot(
        m_ref[0, 0] + m_ref[1, 0], w_ref[0:_FEAT],
        preferred_element_type=jnp.float32)
    acc += jnp.dot(m_ref[0, 1] + m_ref[1, 1], w_ref[_FEAT:2 * _FEAT],
                   preferred_element_type=jnp.float32)
    o_ref[...] = acc


def _dec(mrg_part, w_dec, b1):
    n = mrg_part.shape[2]
    return pl.pallas_call(
        _dec_body,
        grid=(n // _ROWS_TC,),
        in_specs=[
            pl.BlockSpec((2, 2, _ROWS_TC, _FEAT), lambda i: (0, 0, i, 0)),
            pl.BlockSpec((256, 128), lambda i: (0, 0)),
            pl.BlockSpec((1, 128), lambda i: (0, 0)),
        ],
        out_specs=pl.BlockSpec((_ROWS_TC, 128), lambda i: (i, 0)),
        out_shape=jax.ShapeDtypeStruct((n, 128), jnp.float32),
    )(mrg_part, w_dec, b1)


def _edge_merge(tp0, tp1, tn0, tn1, gi2d, si2d, zrows, n_rows, e_pad):
    n_workers = _CORES * _SUBCORES
    tile_e = e_pad // n_workers
    nbt = tile_e // _BATCH
    n_chunks = nbt // _CHUNK
    rpt = n_rows // _SUBCORES
    acc_rows = n_rows + 8

    mesh = plsc.VectorSubcoreMesh(core_axis_name="c", subcore_axis_name="s")

    @functools.partial(
        pl.kernel,
        out_type=jax.ShapeDtypeStruct((4 * n_rows, _FEAT), jnp.float32),
        mesh=mesh,
        scratch_types=[
            pltpu.VMEM_SHARED((acc_rows, _FEAT), jnp.float32),
            pltpu.VMEM((_BATCH,), jnp.int32),
            pltpu.VMEM((_BATCH,), jnp.int32),
            pltpu.VMEM((_BATCH, _FEAT), jnp.float32),
            pltpu.VMEM((_BATCH, _FEAT), jnp.float32),
            pltpu.SemaphoreType.DMA,
            pltpu.SemaphoreType.DMA,
            pltpu.SemaphoreType.DMA,
        ],
    )
    def body(tp0_h, tp1_h, tn0_h, tn1_h, gi_h, si_h, z_h, out_h,
             acc, giv, siv, pbuf, qbuf, psem, qsem, ssem):
        c = lax.axis_index("c")
        s = lax.axis_index("s")
        wtile = c * _SUBCORES + s
        base0 = wtile * tile_e

        def compute():
            def row(j, _):
                for u in range(_FEAT // _LANES):
                    sl = pl.ds(u * _LANES, _LANES)
                    pbuf[j, sl] = jnp.maximum(pbuf[j, sl] - qbuf[j, sl],
                                              0.0)
                return 0

            lax.fori_loop(0, _BATCH, row, 0, unroll=4)

        for k, (tp_h, tn_h) in enumerate(((tp0_h, tn0_h), (tp1_h, tn1_h))):
            pltpu.sync_copy(z_h.at[pl.ds(s * rpt, rpt)],
                            acc.at[pl.ds(s * rpt, rpt)])
            plsc.subcore_barrier()

            def bbody(b, _2, tp_h=tp_h, tn_h=tn_h):
                base = base0 + b * _BATCH
                pltpu.sync_copy(gi_h.at[pl.ds(base, _BATCH)], giv)
                pltpu.sync_copy(si_h.at[pl.ds(base, _BATCH)], siv)
                cp = pltpu.async_copy(tp_h.at[giv], pbuf, psem)
                cq = pltpu.async_copy(tn_h.at[siv], qbuf, qsem)
                cp.wait()
                cq.wait()
                compute()
                pltpu.async_copy(pbuf, acc.at[siv], ssem, add=True).wait()
                return 0

            lax.fori_loop(0, nbt, bbody, 0)
            plsc.subcore_barrier()
            out_base = (c * 2 + k) * n_rows + s * rpt
            pltpu.sync_copy(acc.at[pl.ds(s * rpt, rpt)],
                            out_h.at[pl.ds(out_base, rpt)])
            plsc.subcore_barrier()

    return body(tp0, tp1, tn0, tn1, gi2d, si2d, zrows)


def _pad_edges(idx_g, idx_s, trash):
    e = idx_g.shape[0]
    unit = _CORES * _SUBCORES * _BATCH * _CHUNK
    e_pad = ((e + unit - 1) // unit) * unit
    pad = e_pad - e
    if pad:
        idx_g = jnp.concatenate([idx_g, jnp.zeros((pad,), jnp.int32)])
        idx_s = jnp.concatenate([idx_s, jnp.full((pad,), trash, jnp.int32)])
    return idx_g, idx_s, e_pad


def kernel(obj_x, obj_pos, agent_pos, obj_agent_edge_index, agent_edge_index,
           W_enc, b_enc, W_mrg, b_mrg, W_dec, b_dec):
    f32 = jnp.float32
    n_obj, in_dim = obj_x.shape
    n_ag = agent_pos.shape[0]
    emb = W_enc.shape[1]

    x_cat = jnp.concatenate([obj_x, obj_pos], axis=1)
    xq = jnp.concatenate(
        [jnp.zeros((n_ag, in_dim), f32), agent_pos], axis=1)
    zb = jnp.zeros((emb,), f32)
    n_pad = ((n_ag + 127) // 128) * 128
    zrows = jnp.zeros((n_pad, _FEAT), f32)

    gi1 = obj_agent_edge_index[1].astype(jnp.int32)
    si1 = obj_agent_edge_index[0].astype(jnp.int32)
    gi2 = agent_edge_index[0].astype(jnp.int32)
    si2 = agent_edge_index[1].astype(jnp.int32)
    gi1, si1, e1p = _pad_edges(gi1, si1, n_pad)
    gi2, si2, e2p = _pad_edges(gi2, si2, n_pad)

    p2 = _mm_bias(x_cat, W_enc, b_enc)
    q2 = _mm_bias(xq, W_enc, zb)

    enc_flat = _edge_merge(p2[0], p2[1], q2[0], q2[1], gi1, si1, zrows,
                           n_pad, e1p)
    enc_part = enc_flat.reshape(2, 2, n_pad, _FEAT)[:, :, :n_ag]

    wm_pos_pad = jnp.concatenate(
        [jnp.zeros((in_dim, emb), f32), W_mrg[emb:emb + 2]], axis=0)
    b2_tbl = _mm_bias(xq, wm_pos_pad, zb)
    wma = jnp.concatenate([W_mrg[:emb], wm_pos_pad], axis=0)
    a2_tbl = _mm_enc(enc_part, xq, wma, b_mrg)

    mrg_flat = _edge_merge(a2_tbl[0], a2_tbl[1], b2_tbl[0], b2_tbl[1],
                           gi2, si2, zrows, n_pad, e2p)
    mrg_part = mrg_flat.reshape(2, 2, n_pad, _FEAT)[:, :, :n_ag]

    decoded = _dec(mrg_part, W_dec, b_dec.reshape(1, 128))
    batch = jnp.arange(n_ag, dtype=jnp.int32)
    return decoded, batch


# speedup vs baseline: 1.3827x; 1.3827x over previous
import functools

import jax
import jax.numpy as jnp
from jax import lax
from jax.experimental import pallas as pl
from jax.experimental.pallas import tpu as pltpu
from jax.experimental.pallas import tpu_sc as plsc

_LANES = 16
_SUBCORES = 16
_CORES = 2
_BATCH = 128
_CHUNK = 8
_FEAT = 128
_ROWS_TC = 1000


def _mm_bias_body(x_ref, w_ref, b_ref, o_ref):
    o_ref[0] = (
        jnp.dot(x_ref[...], w_ref[0], preferred_element_type=jnp.float32)
        + b_ref[0, 0][None, :]
    )


def _mm_bias(x, w, b):
    n, k = x.shape
    m = w.shape[1]
    p = m // _FEAT
    return pl.pallas_call(
        _mm_bias_body,
        grid=(p, n // _ROWS_TC),
        in_specs=[
            pl.BlockSpec((_ROWS_TC, k), lambda j, i: (i, 0)),
            pl.BlockSpec((1, k, _FEAT), lambda j, i: (j, 0, 0)),
            pl.BlockSpec((1, 1, _FEAT), lambda j, i: (j, 0, 0)),
        ],
        out_specs=pl.BlockSpec((1, _ROWS_TC, _FEAT), lambda j, i: (j, i, 0)),
        out_shape=jax.ShapeDtypeStruct((p, n, _FEAT), jnp.float32),
    )(x, w.reshape(k, p, _FEAT).transpose(1, 0, 2), b.reshape(p, 1, _FEAT))


def _mm_enc_body(e_ref, x_ref, w_ref, b_ref, o_ref):
    acc = jnp.dot(x_ref[...], w_ref[0, 256:384],
                  preferred_element_type=jnp.float32)
    for u in range(2):
        acc += jnp.dot(e_ref[0, u] + e_ref[1, u],
                       w_ref[0, _FEAT * u:_FEAT * (u + 1)],
                       preferred_element_type=jnp.float32)
    o_ref[0] = acc + b_ref[0, 0][None, :]


def _mm_enc(enc_part, xq, wma, b):
    n = xq.shape[0]
    k = wma.shape[0]
    p = wma.shape[1] // _FEAT
    return pl.pallas_call(
        _mm_enc_body,
        grid=(p, n // _ROWS_TC),
        in_specs=[
            pl.BlockSpec((2, 2, _ROWS_TC, _FEAT), lambda j, i: (0, 0, i, 0)),
            pl.BlockSpec((_ROWS_TC, 128), lambda j, i: (i, 0)),
            pl.BlockSpec((1, k, _FEAT), lambda j, i: (j, 0, 0)),
            pl.BlockSpec((1, 1, _FEAT), lambda j, i: (j, 0, 0)),
        ],
        out_specs=pl.BlockSpec((1, _ROWS_TC, _FEAT), lambda j, i: (j, i, 0)),
        out_shape=jax.ShapeDtypeStruct((p, n, _FEAT), jnp.float32),
    )(enc_part, xq, wma.reshape(k, p, _FEAT).transpose(1, 0, 2),
      b.reshape(p, 1, _FEAT))


def _dec_body(m_ref, w_ref, b_ref, o_ref):
    acc = b_ref[0][None, :] + jnp.dot(
        m_ref[0, 0] + m_ref[1, 0], w_ref[0:_FEAT],
        preferred_element_type=jnp.float32)
    acc += jnp.dot(m_ref[0, 1] + m_ref[1, 1], w_ref[_FEAT:2 * _FEAT],
                   preferred_element_type=jnp.float32)
    o_ref[...] = acc


def _dec(mrg_part, w_dec, b1):
    n = mrg_part.shape[2]
    return pl.pallas_call(
        _dec_body,
        grid=(n // _ROWS_TC,),
        in_specs=[
            pl.BlockSpec((2, 2, _ROWS_TC, _FEAT), lambda i: (0, 0, i, 0)),
            pl.BlockSpec((256, 128), lambda i: (0, 0)),
            pl.BlockSpec((1, 128), lambda i: (0, 0)),
        ],
        out_specs=pl.BlockSpec((_ROWS_TC, 128), lambda i: (i, 0)),
        out_shape=jax.ShapeDtypeStruct((n, 128), jnp.float32),
    )(mrg_part, w_dec, b1)


def _edge_merge(tp0, tp1, tn0, tn1, gi2d, si2d, zrows, n_rows, e_pad):
    n_workers = _CORES * _SUBCORES
    tile_e = e_pad // n_workers
    nbt = tile_e // _BATCH
    n_chunks = nbt // _CHUNK
    rpt = n_rows // _SUBCORES
    acc_rows = n_rows + 8

    mesh = plsc.VectorSubcoreMesh(core_axis_name="c", subcore_axis_name="s")

    @functools.partial(
        pl.kernel,
        out_type=jax.ShapeDtypeStruct((4 * n_rows, _FEAT), jnp.float32),
        mesh=mesh,
        scratch_types=[
            pltpu.VMEM_SHARED((acc_rows, _FEAT), jnp.float32),
            pltpu.VMEM((_BATCH,), jnp.int32),
            pltpu.VMEM((_BATCH,), jnp.int32),
            pltpu.VMEM((_BATCH, _FEAT), jnp.float32),
            pltpu.VMEM((_BATCH, _FEAT), jnp.float32),
            pltpu.SemaphoreType.DMA,
            pltpu.SemaphoreType.DMA,
            pltpu.SemaphoreType.DMA,
        ],
    )
    def body(tp0_h, tp1_h, tn0_h, tn1_h, gi_h, si_h, z_h, out_h,
             acc, giv, siv, pbuf, qbuf, psem, qsem, ssem):
        c = lax.axis_index("c")
        s = lax.axis_index("s")
        wtile = c * _SUBCORES + s
        base0 = wtile * tile_e

        def compute():
            def row(j, _):
                for u in range(_FEAT // _LANES):
                    sl = pl.ds(u * _LANES, _LANES)
                    pbuf[j, sl] = jnp.maximum(pbuf[j, sl] - qbuf[j, sl],
                                              0.0)
                return 0

            lax.fori_loop(0, _BATCH, row, 0)

        for k, (tp_h, tn_h) in enumerate(((tp0_h, tn0_h), (tp1_h, tn1_h))):
            pltpu.sync_copy(z_h.at[pl.ds(s * rpt, rpt)],
                            acc.at[pl.ds(s * rpt, rpt)])
            plsc.subcore_barrier()

            def bbody(b, _2, tp_h=tp_h, tn_h=tn_h):
                base = base0 + b * _BATCH
                pltpu.sync_copy(gi_h.at[pl.ds(base, _BATCH)], giv)
                pltpu.sync_copy(si_h.at[pl.ds(base, _BATCH)], siv)
                cp = pltpu.async_copy(tp_h.at[giv], pbuf, psem)
                cq = pltpu.async_copy(tn_h.at[siv], qbuf, qsem)
                cp.wait()
                cq.wait()
                compute()
                pltpu.sync_copy(pbuf, acc.at[siv], add=True)
                return 0

            lax.fori_loop(0, nbt, bbody, 0)
            plsc.subcore_barrier()
            out_base = (c * 2 + k) * n_rows + s * rpt
            pltpu.sync_copy(acc.at[pl.ds(s * rpt, rpt)],
                            out_h.at[pl.ds(out_base, rpt)])
            plsc.subcore_barrier()

    return body(tp0, tp1, tn0, tn1, gi2d, si2d, zrows)


def _pad_edges(idx_g, idx_s, trash):
    e = idx_g.shape[0]
    unit = _CORES * _SUBCORES * _BATCH * _CHUNK
    e_pad = ((e + unit - 1) // unit) * unit
    pad = e_pad - e
    if pad:
        idx_g = jnp.concatenate([idx_g, jnp.zeros((pad,), jnp.int32)])
        idx_s = jnp.concatenate([idx_s, jnp.full((pad,), trash, jnp.int32)])
    return idx_g, idx_s, e_pad


def kernel(obj_x, obj_pos, agent_pos, obj_agent_edge_index, agent_edge_index,
           W_enc, b_enc, W_mrg, b_mrg, W_dec, b_dec):
    f32 = jnp.float32
    n_obj, in_dim = obj_x.shape
    n_ag = agent_pos.shape[0]
    emb = W_enc.shape[1]

    x_cat = jnp.concatenate([obj_x, obj_pos], axis=1)
    xq = jnp.concatenate(
        [jnp.zeros((n_ag, in_dim), f32), agent_pos], axis=1)
    zb = jnp.zeros((emb,), f32)
    n_pad = ((n_ag + 127) // 128) * 128
    zrows = jnp.zeros((n_pad, _FEAT), f32)

    gi1 = obj_agent_edge_index[1].astype(jnp.int32)
    si1 = obj_agent_edge_index[0].astype(jnp.int32)
    gi2 = agent_edge_index[0].astype(jnp.int32)
    si2 = agent_edge_index[1].astype(jnp.int32)
    gi1, si1, e1p = _pad_edges(gi1, si1, n_pad)
    gi2, si2, e2p = _pad_edges(gi2, si2, n_pad)

    p2 = _mm_bias(x_cat, W_enc, b_enc)
    q2 = _mm_bias(xq, W_enc, zb)

    enc_flat = _edge_merge(p2[0], p2[1], q2[0], q2[1], gi1, si1, zrows,
                           n_pad, e1p)
    enc_part = enc_flat.reshape(2, 2, n_pad, _FEAT)[:, :, :n_ag]

    wm_pos_pad = jnp.concatenate(
        [jnp.zeros((in_dim, emb), f32), W_mrg[emb:emb + 2]], axis=0)
    b2_tbl = _mm_bias(xq, wm_pos_pad, zb)
    wma = jnp.concatenate([W_mrg[:emb], wm_pos_pad], axis=0)
    a2_tbl = _mm_enc(enc_part, xq, wma, b_mrg)

    mrg_flat = _edge_merge(a2_tbl[0], a2_tbl[1], b2_tbl[0], b2_tbl[1],
                           gi2, si2, zrows, n_pad, e2p)
    mrg_part = mrg_flat.reshape(2, 2, n_pad, _FEAT)[:, :, :n_ag]

    decoded = _dec(mrg_part, W_dec, b_dec.reshape(1, 128))
    batch = jnp.arange(n_ag, dtype=jnp.int32)
    return decoded, batch


# speedup vs baseline: 1.6562x; 1.1978x over previous
import functools

import jax
import jax.numpy as jnp
from jax import lax
from jax.experimental import pallas as pl
from jax.experimental.pallas import tpu as pltpu
from jax.experimental.pallas import tpu_sc as plsc

_LANES = 16
_SUBCORES = 16
_CORES = 2
_BATCH = 128
_CHUNK = 8
_FEAT = 128
_ROWS_TC = 1000


def _mm_bias_body(x_ref, w_ref, b_ref, o_ref):
    o_ref[0] = (
        jnp.dot(x_ref[...], w_ref[0], preferred_element_type=jnp.float32)
        + b_ref[0, 0][None, :]
    )


def _mm_bias(x, w, b):
    n, k = x.shape
    m = w.shape[1]
    p = m // _FEAT
    return pl.pallas_call(
        _mm_bias_body,
        grid=(p, n // _ROWS_TC),
        in_specs=[
            pl.BlockSpec((_ROWS_TC, k), lambda j, i: (i, 0)),
            pl.BlockSpec((1, k, _FEAT), lambda j, i: (j, 0, 0)),
            pl.BlockSpec((1, 1, _FEAT), lambda j, i: (j, 0, 0)),
        ],
        out_specs=pl.BlockSpec((1, _ROWS_TC, _FEAT), lambda j, i: (j, i, 0)),
        out_shape=jax.ShapeDtypeStruct((p, n, _FEAT), jnp.float32),
    )(x, w.reshape(k, p, _FEAT).transpose(1, 0, 2), b.reshape(p, 1, _FEAT))


def _mm_enc_body(e_ref, x_ref, w_ref, b_ref, o_ref):
    acc = jnp.dot(x_ref[...], w_ref[0, 256:384],
                  preferred_element_type=jnp.float32)
    for u in range(2):
        acc += jnp.dot(e_ref[0, u] + e_ref[1, u],
                       w_ref[0, _FEAT * u:_FEAT * (u + 1)],
                       preferred_element_type=jnp.float32)
    o_ref[0] = acc + b_ref[0, 0][None, :]


def _mm_enc(enc_part, xq, wma, b):
    n = xq.shape[0]
    k = wma.shape[0]
    p = wma.shape[1] // _FEAT
    return pl.pallas_call(
        _mm_enc_body,
        grid=(p, n // _ROWS_TC),
        in_specs=[
            pl.BlockSpec((2, 2, _ROWS_TC, _FEAT), lambda j, i: (0, 0, i, 0)),
            pl.BlockSpec((_ROWS_TC, 128), lambda j, i: (i, 0)),
            pl.BlockSpec((1, k, _FEAT), lambda j, i: (j, 0, 0)),
            pl.BlockSpec((1, 1, _FEAT), lambda j, i: (j, 0, 0)),
        ],
        out_specs=pl.BlockSpec((1, _ROWS_TC, _FEAT), lambda j, i: (j, i, 0)),
        out_shape=jax.ShapeDtypeStruct((p, n, _FEAT), jnp.float32),
    )(enc_part, xq, wma.reshape(k, p, _FEAT).transpose(1, 0, 2),
      b.reshape(p, 1, _FEAT))


def _dec_body(m_ref, w_ref, b_ref, o_ref):
    acc = b_ref[0][None, :] + jnp.dot(
        m_ref[0, 0] + m_ref[1, 0], w_ref[0:_FEAT],
        preferred_element_type=jnp.float32)
    acc += jnp.dot(m_ref[0, 1] + m_ref[1, 1], w_ref[_FEAT:2 * _FEAT],
                   preferred_element_type=jnp.float32)
    o_ref[...] = acc


def _dec(mrg_part, w_dec, b1):
    n = mrg_part.shape[2]
    return pl.pallas_call(
        _dec_body,
        grid=(n // _ROWS_TC,),
        in_specs=[
            pl.BlockSpec((2, 2, _ROWS_TC, _FEAT), lambda i: (0, 0, i, 0)),
            pl.BlockSpec((256, 128), lambda i: (0, 0)),
            pl.BlockSpec((1, 128), lambda i: (0, 0)),
        ],
        out_specs=pl.BlockSpec((_ROWS_TC, 128), lambda i: (i, 0)),
        out_shape=jax.ShapeDtypeStruct((n, 128), jnp.float32),
    )(mrg_part, w_dec, b1)


def _edge_merge(tp0, tp1, tn0, tn1, gi2d, si2d, zrows, n_rows, e_pad):
    n_workers = _CORES * _SUBCORES
    tile_e = e_pad // n_workers
    nbt = tile_e // _BATCH
    n_chunks = nbt // _CHUNK
    rpt = n_rows // _SUBCORES
    acc_rows = n_rows + 8

    mesh = plsc.VectorSubcoreMesh(core_axis_name="c", subcore_axis_name="s")

    @functools.partial(
        pl.kernel,
        out_type=jax.ShapeDtypeStruct((4 * n_rows, _FEAT), jnp.float32),
        mesh=mesh,
        scratch_types=[
            pltpu.VMEM_SHARED((acc_rows, _FEAT), jnp.float32),
            pltpu.VMEM((_BATCH,), jnp.int32),
            pltpu.VMEM((_BATCH,), jnp.int32),
            pltpu.VMEM((_BATCH, _FEAT), jnp.float32),
            pltpu.VMEM((_BATCH, _FEAT), jnp.float32),
            pltpu.SemaphoreType.DMA,
            pltpu.SemaphoreType.DMA,
        ],
    )
    def body(tp0_h, tp1_h, tn0_h, tn1_h, gi_h, si_h, z_h, out_h,
             acc, giv, siv, pbuf, qbuf, psem, qsem):
        c = lax.axis_index("c")
        s = lax.axis_index("s")
        wtile = c * _SUBCORES + s
        base0 = wtile * tile_e

        def compute():
            def row(j, _):
                for u in range(_FEAT // _LANES):
                    sl = pl.ds(u * _LANES, _LANES)
                    pbuf[j, sl] = jnp.maximum(pbuf[j, sl] - qbuf[j, sl],
                                              0.0)
                return 0

            lax.fori_loop(0, _BATCH, row, 0)

        for k, (tp_h, tn_h) in enumerate(((tp0_h, tn0_h), (tp1_h, tn1_h))):
            pltpu.sync_copy(z_h.at[pl.ds(s * rpt, rpt)],
                            acc.at[pl.ds(s * rpt, rpt)])
            plsc.subcore_barrier()

            def bbody(b, _2, tp_h=tp_h, tn_h=tn_h):
                base = base0 + b * _BATCH
                pltpu.sync_copy(gi_h.at[pl.ds(base, _BATCH)], giv)
                pltpu.sync_copy(si_h.at[pl.ds(base, _BATCH)], siv)
                cp = pltpu.async_copy(tp_h.at[giv], pbuf, psem)
                cq = pltpu.async_copy(tn_h.at[siv], qbuf, qsem)
                cp.wait()
                cq.wait()
                compute()
                pltpu.sync_copy(pbuf, acc.at[siv], add=True)
                return 0

            lax.fori_loop(0, nbt, bbody, 0)
            plsc.subcore_barrier()
            out_base = (c * 2 + k) * n_rows + s * rpt
            pltpu.sync_copy(acc.at[pl.ds(s * rpt, rpt)],
                            out_h.at[pl.ds(out_base, rpt)])
            plsc.subcore_barrier()

    return body(tp0, tp1, tn0, tn1, gi2d, si2d, zrows)


def _pad_edges(idx_g, idx_s, trash):
    e = idx_g.shape[0]
    unit = _CORES * _SUBCORES * _BATCH
    e_pad = ((e + unit - 1) // unit) * unit
    pad = e_pad - e
    if pad:
        idx_g = jnp.concatenate([idx_g, jnp.zeros((pad,), jnp.int32)])
        idx_s = jnp.concatenate([idx_s, jnp.full((pad,), trash, jnp.int32)])
    return idx_g, idx_s, e_pad


def kernel(obj_x, obj_pos, agent_pos, obj_agent_edge_index, agent_edge_index,
           W_enc, b_enc, W_mrg, b_mrg, W_dec, b_dec):
    f32 = jnp.float32
    n_obj, in_dim = obj_x.shape
    n_ag = agent_pos.shape[0]
    emb = W_enc.shape[1]

    x_cat = jnp.concatenate([obj_x, obj_pos], axis=1)
    xq = jnp.concatenate(
        [jnp.zeros((n_ag, in_dim), f32), agent_pos], axis=1)
    zb = jnp.zeros((emb,), f32)
    n_pad = ((n_ag + 127) // 128) * 128
    zrows = jnp.zeros((n_pad, _FEAT), f32)

    gi1 = obj_agent_edge_index[1].astype(jnp.int32)
    si1 = obj_agent_edge_index[0].astype(jnp.int32)
    gi2 = agent_edge_index[0].astype(jnp.int32)
    si2 = agent_edge_index[1].astype(jnp.int32)
    gi1, si1, e1p = _pad_edges(gi1, si1, n_pad)
    gi2, si2, e2p = _pad_edges(gi2, si2, n_pad)

    p2 = _mm_bias(x_cat, W_enc, b_enc)
    q2 = _mm_bias(xq, W_enc, zb)

    enc_flat = _edge_merge(p2[0], p2[1], q2[0], q2[1], gi1, si1, zrows,
                           n_pad, e1p)
    enc_part = enc_flat.reshape(2, 2, n_pad, _FEAT)[:, :, :n_ag]

    wm_pos_pad = jnp.concatenate(
        [jnp.zeros((in_dim, emb), f32), W_mrg[emb:emb + 2]], axis=0)
    b2_tbl = _mm_bias(xq, wm_pos_pad, zb)
    wma = jnp.concatenate([W_mrg[:emb], wm_pos_pad], axis=0)
    a2_tbl = _mm_enc(enc_part, xq, wma, b_mrg)

    mrg_flat = _edge_merge(a2_tbl[0], a2_tbl[1], b2_tbl[0], b2_tbl[1],
                           gi2, si2, zrows, n_pad, e2p)
    mrg_part = mrg_flat.reshape(2, 2, n_pad, _FEAT)[:, :, :n_ag]

    decoded = _dec(mrg_part, W_dec, b_dec.reshape(1, 128))
    batch = jnp.arange(n_ag, dtype=jnp.int32)
    return decoded, batch
